# Initial kernel scaffold; baseline (speedup 1.0000x reference)
import jax
import jax.numpy as jnp
from jax.experimental import pallas as pl


def kernel(x, edge_index, W1, b1, W2, b2):
    raise NotImplementedError("write your pallas kernel here")


# speedup vs baseline: 15.7598x; 15.7598x over previous
import functools

import jax
import jax.numpy as jnp
from jax import lax
from jax.experimental import pallas as pl
from jax.experimental.pallas import tpu as pltpu
from jax.experimental.pallas import tpu_sc as plsc

N = 10000
E = 320000
D = 128
DH = 64
NC = 2
NS = 16
K = 128
NW = NC * NS
CH = 80
GSZ = 16
GCH = CH // GSZ
EPT = CH * K
EPAD = NW * EPT
NPAD = 10240
RPT = NPAD // NS
GRID = 10
BLK = NPAD // GRID


def _sc_mesh():
    return plsc.VectorSubcoreMesh(core_axis_name="c", subcore_axis_name="s")


def _deg_body(src_hbm, dst_hbm, zvec_hbm, deg_hbm, acc, idx_v, ones_v, sem):
    c = lax.axis_index("c")
    s = lax.axis_index("s")
    for i in range(K // 16):
        ones_v[pl.ds(i * 16, 16)] = jnp.full((16,), 1.0, jnp.float32)
    pltpu.sync_copy(zvec_hbm, acc.at[pl.ds(s * RPT, RPT)])
    plsc.subcore_barrier()
    @pl.when(c == 0)
    def _():
        pltpu.sync_copy(src_hbm.at[s], idx_v)

    @pl.when(c == 1)
    def _():
        pltpu.sync_copy(dst_hbm.at[s], idx_v)

    def body(j, carry):
        pltpu.sync_copy(ones_v, acc.at[idx_v.at[j]], add=True)
        return carry

    lax.fori_loop(0, 2 * CH, body, 0)
    plsc.subcore_barrier()
    pltpu.sync_copy(acc.at[pl.ds(s * RPT, RPT)],
                    deg_hbm.at[c, pl.ds(s * RPT, RPT)])


def _make_deg_kernel():
    return pl.kernel(
        _deg_body,
        out_type=jax.ShapeDtypeStruct((NC, NPAD), jnp.float32),
        mesh=_sc_mesh(),
        scratch_types=[
            pltpu.VMEM_SHARED((NPAD,), jnp.float32),
            pltpu.VMEM((2 * CH, K), jnp.int32),
            pltpu.VMEM((K,), jnp.float32),
            pltpu.SemaphoreType.DMA,
        ],
    )


def _main_body(y_hbm, src_hbm, dst_hbm, nd_hbm, zrows_hbm, zvec_hbm,
               agg_hbm, cpart_hbm,
               agg_sp, nd_sp, c_sp,
               src_v, dst_v, rows_a, rows_b, vals_a, vals_b,
               gsem_a, gsem_b, ssem_a, ssem_b, vsem_a, vsem_b, csem_a, csem_b):
    c = lax.axis_index("c")
    s = lax.axis_index("s")
    w = c * NS + s
    row0 = s * RPT
    pltpu.sync_copy(zrows_hbm, agg_sp.at[pl.ds(row0, RPT)])
    pltpu.sync_copy(zvec_hbm, c_sp.at[pl.ds(row0, RPT)])
    pltpu.sync_copy(nd_hbm.at[pl.ds(row0, RPT)], nd_sp.at[pl.ds(row0, RPT)])
    plsc.subcore_barrier()

    rows = (rows_a, rows_b)
    vals = (vals_a, vals_b)
    gsem = (gsem_a, gsem_b)
    ssem = (ssem_a, ssem_b)
    vsem = (vsem_a, vsem_b)
    csem = (csem_a, csem_b)

    def group(g, carry):
        pltpu.sync_copy(src_hbm.at[w, pl.ds(g * GSZ, GSZ)], src_v)
        pltpu.sync_copy(dst_hbm.at[w, pl.ds(g * GSZ, GSZ)], dst_v)

        scat = [None, None]
        cscat = [None, None]
        for j in range(GSZ):
            sl = j % 2
            if scat[sl] is not None:
                scat[sl].wait()
            pltpu.async_copy(y_hbm.at[src_v.at[j]], rows[sl], gsem[sl]).wait()
            scat[sl] = pltpu.async_copy(rows[sl], agg_sp.at[dst_v.at[j]],
                                        ssem[sl], add=True)
            if cscat[sl] is not None:
                cscat[sl].wait()
            pltpu.async_copy(nd_sp.at[dst_v.at[j]], vals[sl], vsem[sl]).wait()
            cscat[sl] = pltpu.async_copy(vals[sl], c_sp.at[src_v.at[j]],
                                         csem[sl], add=True)

        for sl in range(2):
            if scat[sl] is not None:
                scat[sl].wait()
            if cscat[sl] is not None:
                cscat[sl].wait()
        return carry

    lax.fori_loop(0, GCH, group, 0)
    plsc.subcore_barrier()
    pltpu.sync_copy(agg_sp.at[pl.ds(row0, RPT)],
                    agg_hbm.at[c, pl.ds(row0, RPT)])
    pltpu.sync_copy(c_sp.at[pl.ds(row0, RPT)],
                    cpart_hbm.at[c, pl.ds(row0, RPT)])


def _make_main_kernel():
    return pl.kernel(
        _main_body,
        out_type=(
            jax.ShapeDtypeStruct((NC, NPAD, D), jnp.float32),
            jax.ShapeDtypeStruct((NC, NPAD), jnp.float32),
        ),
        mesh=_sc_mesh(),
        scratch_types=[
            pltpu.VMEM_SHARED((NPAD, D), jnp.float32),
            pltpu.VMEM_SHARED((NPAD,), jnp.float32),
            pltpu.VMEM_SHARED((NPAD,), jnp.float32),
            pltpu.VMEM((GSZ, K), jnp.int32),
            pltpu.VMEM((GSZ, K), jnp.int32),
            pltpu.VMEM((K, D), jnp.float32),
            pltpu.VMEM((K, D), jnp.float32),
            pltpu.VMEM((K,), jnp.float32),
            pltpu.VMEM((K,), jnp.float32),
            pltpu.SemaphoreType.DMA,
            pltpu.SemaphoreType.DMA,
            pltpu.SemaphoreType.DMA,
            pltpu.SemaphoreType.DMA,
            pltpu.SemaphoreType.DMA,
            pltpu.SemaphoreType.DMA,
            pltpu.SemaphoreType.DMA,
            pltpu.SemaphoreType.DMA,
        ],
    )


def _prep_kernel(x_ref, w1_ref, dego_ref, degi_ref,
                 y_ref, ns_ref, nd_ref):
    ns = lax.rsqrt(jnp.maximum(dego_ref[...], 1.0))
    nd = lax.rsqrt(jnp.maximum(degi_ref[...], 1.0))
    y_ref[...] = jnp.dot(x_ref[...], w1_ref[...],
                         preferred_element_type=jnp.float32) * ns
    ns_ref[...] = ns
    nd_ref[...] = nd


def _prep(xp, W1, dego, degi):
    return pl.pallas_call(
        _prep_kernel,
        grid=(GRID,),
        in_specs=[
            pl.BlockSpec((BLK, D), lambda i: (i, 0)),
            pl.BlockSpec((D, D), lambda i: (0, 0)),
            pl.BlockSpec((BLK, 1), lambda i: (i, 0)),
            pl.BlockSpec((BLK, 1), lambda i: (i, 0)),
        ],
        out_specs=(
            pl.BlockSpec((BLK, D), lambda i: (i, 0)),
            pl.BlockSpec((BLK, 1), lambda i: (i, 0)),
            pl.BlockSpec((BLK, 1), lambda i: (i, 0)),
        ),
        out_shape=(
            jax.ShapeDtypeStruct((NPAD, D), jnp.float32),
            jax.ShapeDtypeStruct((NPAD, 1), jnp.float32),
            jax.ShapeDtypeStruct((NPAD, 1), jnp.float32),
        ),
    )(xp, W1, dego, degi)


def _final_kernel(agg_ref, nd_ref, ns_ref, cp_ref, b1_ref, w2_ref, b2_ref,
                  out_ref, acc_ref):
    i = pl.program_id(0)

    @pl.when(i == 0)
    def _():
        acc_ref[...] = jnp.zeros_like(acc_ref)

    nd = nd_ref[...]
    h = jnp.maximum((agg_ref[0] + agg_ref[1]) * nd + b1_ref[...], 0.0)
    rowid = lax.broadcasted_iota(jnp.int32, (BLK, 1), 0) + i * BLK
    mask = (rowid < N).astype(jnp.float32)
    w = ns_ref[...] * (cp_ref[0] + cp_ref[1]) * mask
    acc_ref[...] += jnp.sum(h * w, axis=0, keepdims=True)

    @pl.when(i == GRID - 1)
    def _():
        pooled = acc_ref[0:1, :] * (1.0 / N)
        out_ref[...] = jnp.dot(pooled, w2_ref[...],
                               preferred_element_type=jnp.float32) + b2_ref[...]


def _final(agg, ndp, nsp, cp2, b1r, W2, b2r):
    return pl.pallas_call(
        _final_kernel,
        grid=(GRID,),
        in_specs=[
            pl.BlockSpec((NC, BLK, D), lambda i: (0, i, 0)),
            pl.BlockSpec((BLK, 1), lambda i: (i, 0)),
            pl.BlockSpec((BLK, 1), lambda i: (i, 0)),
            pl.BlockSpec((NC, BLK, 1), lambda i: (0, i, 0)),
            pl.BlockSpec((1, D), lambda i: (0, 0)),
            pl.BlockSpec((D, D), lambda i: (0, 0)),
            pl.BlockSpec((1, D), lambda i: (0, 0)),
        ],
        out_specs=pl.BlockSpec((1, D), lambda i: (0, 0)),
        out_shape=jax.ShapeDtypeStruct((1, D), jnp.float32),
        scratch_shapes=[pltpu.VMEM((1, D), jnp.float32)],
    )(agg, ndp, nsp, cp2, b1r, W2, b2r)


def kernel(x, edge_index, W1, b1, W2, b2):
    src = edge_index[0]
    dst = edge_index[1]
    pad = N + (jnp.arange(EPAD - E, dtype=jnp.int32) % (NPAD - N))
    srcp = jnp.concatenate([src, pad])
    dstp = jnp.concatenate([dst, pad])
    src3 = srcp.reshape(NW, CH, K)
    dst3 = dstp.reshape(NW, CH, K)
    xp = jnp.pad(x, ((0, NPAD - N), (0, 0)))
    zvec = jnp.zeros((RPT,), jnp.float32)
    zrows = jnp.zeros((RPT, D), jnp.float32)

    deg2 = _make_deg_kernel()(srcp.reshape(NS, 2 * CH, K),
                              dstp.reshape(NS, 2 * CH, K), zvec)
    dego = deg2[0].reshape(NPAD, 1)
    degi = deg2[1].reshape(NPAD, 1)

    y_full, nsp, ndp = _prep(xp, W1, dego, degi)

    agg, cpart = _make_main_kernel()(y_full, src3, dst3,
                                     ndp.reshape(NPAD), zrows, zvec)

    out = _final(agg, ndp, nsp, cpart.reshape(NC, NPAD, 1),
                 b1.reshape(1, D), W2, b2.reshape(1, D))
    return out


# speedup vs baseline: 18.4080x; 1.1680x over previous
import functools

import jax
import jax.numpy as jnp
from jax import lax
from jax.experimental import pallas as pl
from jax.experimental.pallas import tpu as pltpu
from jax.experimental.pallas import tpu_sc as plsc

N = 10000
E = 320000
D = 128
DH = 64
NC = 2
NS = 16
K = 128
NW = NC * NS
CH = 80
GSZ = 16
GCH = CH // GSZ
EPT = CH * K
EPAD = NW * EPT
NPAD = 10240
RPT = NPAD // NS
GRID = 10
BLK = NPAD // GRID


def _sc_mesh():
    return plsc.VectorSubcoreMesh(core_axis_name="c", subcore_axis_name="s")


def _deg_body(src_hbm, dst_hbm, zvec_hbm, deg_hbm, acc, idx_v, ones_v, sem):
    c = lax.axis_index("c")
    s = lax.axis_index("s")
    for i in range(K // 16):
        ones_v[pl.ds(i * 16, 16)] = jnp.full((16,), 1.0, jnp.float32)
    pltpu.sync_copy(zvec_hbm, acc.at[pl.ds(s * RPT, RPT)])
    plsc.subcore_barrier()
    @pl.when(c == 0)
    def _():
        pltpu.sync_copy(src_hbm.at[s], idx_v)

    @pl.when(c == 1)
    def _():
        pltpu.sync_copy(dst_hbm.at[s], idx_v)

    for wave in range((2 * CH) // 16):
        descs = [pltpu.async_copy(ones_v, acc.at[idx_v.at[wave * 16 + i]],
                                  sem, add=True) for i in range(16)]
        for dsc in descs:
            dsc.wait()
    plsc.subcore_barrier()
    pltpu.sync_copy(acc.at[pl.ds(s * RPT, RPT)],
                    deg_hbm.at[c, pl.ds(s * RPT, RPT)])


def _make_deg_kernel():
    return pl.kernel(
        _deg_body,
        out_type=jax.ShapeDtypeStruct((NC, NPAD), jnp.float32),
        mesh=_sc_mesh(),
        scratch_types=[
            pltpu.VMEM_SHARED((NPAD,), jnp.float32),
            pltpu.VMEM((2 * CH, K), jnp.int32),
            pltpu.VMEM((K,), jnp.float32),
            pltpu.SemaphoreType.DMA,
        ],
    )


def _main_body(y_hbm, src_hbm, dst_hbm, nd_hbm, zrows_hbm, zvec_hbm,
               agg_hbm, cpart_hbm,
               agg_sp, nd_sp, c_sp,
               src_v, dst_v, rows_a, rows_b, vals_a, vals_b,
               gsem_a, gsem_b, ssem_a, ssem_b, vsem_a, vsem_b, csem_a, csem_b):
    c = lax.axis_index("c")
    s = lax.axis_index("s")
    w = c * NS + s
    row0 = s * RPT
    pltpu.sync_copy(zrows_hbm, agg_sp.at[pl.ds(row0, RPT)])
    pltpu.sync_copy(zvec_hbm, c_sp.at[pl.ds(row0, RPT)])
    pltpu.sync_copy(nd_hbm.at[pl.ds(row0, RPT)], nd_sp.at[pl.ds(row0, RPT)])
    plsc.subcore_barrier()

    rows = (rows_a, rows_b)
    vals = (vals_a, vals_b)
    gsem = (gsem_a, gsem_b)
    ssem = (ssem_a, ssem_b)
    vsem = (vsem_a, vsem_b)
    csem = (csem_a, csem_b)

    def group(g, carry):
        pltpu.sync_copy(src_hbm.at[w, pl.ds(g * GSZ, GSZ)], src_v)
        pltpu.sync_copy(dst_hbm.at[w, pl.ds(g * GSZ, GSZ)], dst_v)

        scat = [None, None]
        cscat = [None, None]
        gat = [None, None]
        cgat = [None, None]
        gat[0] = pltpu.async_copy(y_hbm.at[src_v.at[0]], rows[0], gsem[0])
        cgat[0] = pltpu.async_copy(nd_sp.at[dst_v.at[0]], vals[0], vsem[0])
        for j in range(GSZ):
            sl = j % 2
            osl = 1 - sl
            if j + 1 < GSZ:
                if scat[osl] is not None:
                    scat[osl].wait()
                if cscat[osl] is not None:
                    cscat[osl].wait()
                gat[osl] = pltpu.async_copy(y_hbm.at[src_v.at[j + 1]],
                                            rows[osl], gsem[osl])
                cgat[osl] = pltpu.async_copy(nd_sp.at[dst_v.at[j + 1]],
                                             vals[osl], vsem[osl])
            gat[sl].wait()
            scat[sl] = pltpu.async_copy(rows[sl], agg_sp.at[dst_v.at[j]],
                                        ssem[sl], add=True)
            cgat[sl].wait()
            cscat[sl] = pltpu.async_copy(vals[sl], c_sp.at[src_v.at[j]],
                                         csem[sl], add=True)

        for sl in range(2):
            if scat[sl] is not None:
                scat[sl].wait()
            if cscat[sl] is not None:
                cscat[sl].wait()
        return carry

    lax.fori_loop(0, GCH, group, 0)
    plsc.subcore_barrier()
    pltpu.sync_copy(agg_sp.at[pl.ds(row0, RPT)],
                    agg_hbm.at[c, pl.ds(row0, RPT)])
    pltpu.sync_copy(c_sp.at[pl.ds(row0, RPT)],
                    cpart_hbm.at[c, pl.ds(row0, RPT)])


def _make_main_kernel():
    return pl.kernel(
        _main_body,
        out_type=(
            jax.ShapeDtypeStruct((NC, NPAD, D), jnp.float32),
            jax.ShapeDtypeStruct((NC, NPAD), jnp.float32),
        ),
        mesh=_sc_mesh(),
        scratch_types=[
            pltpu.VMEM_SHARED((NPAD, D), jnp.float32),
            pltpu.VMEM_SHARED((NPAD,), jnp.float32),
            pltpu.VMEM_SHARED((NPAD,), jnp.float32),
            pltpu.VMEM((GSZ, K), jnp.int32),
            pltpu.VMEM((GSZ, K), jnp.int32),
            pltpu.VMEM((K, D), jnp.float32),
            pltpu.VMEM((K, D), jnp.float32),
            pltpu.VMEM((K,), jnp.float32),
            pltpu.VMEM((K,), jnp.float32),
            pltpu.SemaphoreType.DMA,
            pltpu.SemaphoreType.DMA,
            pltpu.SemaphoreType.DMA,
            pltpu.SemaphoreType.DMA,
            pltpu.SemaphoreType.DMA,
            pltpu.SemaphoreType.DMA,
            pltpu.SemaphoreType.DMA,
            pltpu.SemaphoreType.DMA,
        ],
    )


def _prep_kernel(x_ref, w1_ref, dego_ref, degi_ref,
                 y_ref, ns_ref, nd_ref):
    ns = lax.rsqrt(jnp.maximum(dego_ref[...], 1.0))
    nd = lax.rsqrt(jnp.maximum(degi_ref[...], 1.0))
    y_ref[...] = jnp.dot(x_ref[...], w1_ref[...],
                         preferred_element_type=jnp.float32) * ns
    ns_ref[...] = ns
    nd_ref[...] = nd


def _prep(xp, W1, dego, degi):
    return pl.pallas_call(
        _prep_kernel,
        grid=(GRID,),
        in_specs=[
            pl.BlockSpec((BLK, D), lambda i: (i, 0)),
            pl.BlockSpec((D, D), lambda i: (0, 0)),
            pl.BlockSpec((BLK, 1), lambda i: (i, 0)),
            pl.BlockSpec((BLK, 1), lambda i: (i, 0)),
        ],
        out_specs=(
            pl.BlockSpec((BLK, D), lambda i: (i, 0)),
            pl.BlockSpec((BLK, 1), lambda i: (i, 0)),
            pl.BlockSpec((BLK, 1), lambda i: (i, 0)),
        ),
        out_shape=(
            jax.ShapeDtypeStruct((NPAD, D), jnp.float32),
            jax.ShapeDtypeStruct((NPAD, 1), jnp.float32),
            jax.ShapeDtypeStruct((NPAD, 1), jnp.float32),
        ),
    )(xp, W1, dego, degi)


def _final_kernel(agg_ref, nd_ref, ns_ref, cp_ref, b1_ref, w2_ref, b2_ref,
                  out_ref, acc_ref):
    i = pl.program_id(0)

    @pl.when(i == 0)
    def _():
        acc_ref[...] = jnp.zeros_like(acc_ref)

    nd = nd_ref[...]
    h = jnp.maximum((agg_ref[0] + agg_ref[1]) * nd + b1_ref[...], 0.0)
    rowid = lax.broadcasted_iota(jnp.int32, (BLK, 1), 0) + i * BLK
    mask = (rowid < N).astype(jnp.float32)
    w = ns_ref[...] * (cp_ref[0] + cp_ref[1]) * mask
    acc_ref[...] += jnp.sum(h * w, axis=0, keepdims=True)

    @pl.when(i == GRID - 1)
    def _():
        pooled = acc_ref[0:1, :] * (1.0 / N)
        out_ref[...] = jnp.dot(pooled, w2_ref[...],
                               preferred_element_type=jnp.float32) + b2_ref[...]


def _final(agg, ndp, nsp, cp2, b1r, W2, b2r):
    return pl.pallas_call(
        _final_kernel,
        grid=(GRID,),
        in_specs=[
            pl.BlockSpec((NC, BLK, D), lambda i: (0, i, 0)),
            pl.BlockSpec((BLK, 1), lambda i: (i, 0)),
            pl.BlockSpec((BLK, 1), lambda i: (i, 0)),
            pl.BlockSpec((NC, BLK, 1), lambda i: (0, i, 0)),
            pl.BlockSpec((1, D), lambda i: (0, 0)),
            pl.BlockSpec((D, D), lambda i: (0, 0)),
            pl.BlockSpec((1, D), lambda i: (0, 0)),
        ],
        out_specs=pl.BlockSpec((1, D), lambda i: (0, 0)),
        out_shape=jax.ShapeDtypeStruct((1, D), jnp.float32),
        scratch_shapes=[pltpu.VMEM((1, D), jnp.float32)],
    )(agg, ndp, nsp, cp2, b1r, W2, b2r)


def kernel(x, edge_index, W1, b1, W2, b2):
    src = edge_index[0]
    dst = edge_index[1]
    pad = N + (jnp.arange(EPAD - E, dtype=jnp.int32) % (NPAD - N))
    srcp = jnp.concatenate([src, pad])
    dstp = jnp.concatenate([dst, pad])
    src3 = srcp.reshape(NW, CH, K)
    dst3 = dstp.reshape(NW, CH, K)
    xp = jnp.pad(x, ((0, NPAD - N), (0, 0)))
    zvec = jnp.zeros((RPT,), jnp.float32)
    zrows = jnp.zeros((RPT, D), jnp.float32)

    deg2 = _make_deg_kernel()(srcp.reshape(NS, 2 * CH, K),
                              dstp.reshape(NS, 2 * CH, K), zvec)
    dego = deg2[0].reshape(NPAD, 1)
    degi = deg2[1].reshape(NPAD, 1)

    y_full, nsp, ndp = _prep(xp, W1, dego, degi)

    agg, cpart = _make_main_kernel()(y_full, src3, dst3,
                                     ndp.reshape(NPAD), zrows, zvec)

    out = _final(agg, ndp, nsp, cpart.reshape(NC, NPAD, 1),
                 b1.reshape(1, D), W2, b2.reshape(1, D))
    return out
